# baseline (speedup 1.0000x reference)
import functools
import math

import jax
import jax.numpy as jnp
from jax import lax
from jax.experimental import pallas as pl
from jax.experimental.pallas import tpu as pltpu
from jax.experimental.pallas import tpu_sc as plsc

N_NODES = 100000
PE_DIM = 64
NE2_RAW = 6400000
NC, NS, L = 2, 16, 16
NW = NC * NS
CHUNK = 2048
CHUNKS_PER_W = 98
PER_W = CHUNK * CHUNKS_PER_W
NE2 = NW * PER_W
PAD = NE2 - NE2_RAW

NB_USED = 78125000
DUMMY = NB_USED
NB = NB_USED + 8

NPAD_NODES = 102400
DEG_SLICE = NPAD_NODES // NS
PE_BLK = 2048

_MESH = plsc.VectorSubcoreMesh(core_axis_name="c", subcore_axis_name="s")


def _keys(rows_v, cols_v, o):
    r = rows_v[pl.ds(o, L)]
    c = cols_v[pl.ds(o, L)]
    t = r * 32 + c
    low7 = lax.bitwise_and(t, jnp.int32(127))
    bkt = r * 781 + lax.shift_right_logical(t, jnp.int32(7))
    return low7, bkt


def _worker_base():
    w = lax.axis_index("c") * NS + lax.axis_index("s")
    return w, w * PER_W


@functools.partial(
    pl.kernel,
    out_type=jax.ShapeDtypeStruct((NB,), jnp.int32),
    mesh=_MESH,
    scratch_types=[
        pltpu.VMEM((CHUNK,), jnp.int32),
        pltpu.VMEM((CHUNK,), jnp.int32),
        pltpu.VMEM((CHUNK,), jnp.int32),
        pltpu.VMEM((16, 128), jnp.int32),
        pltpu.VMEM((16, 128), jnp.int32),
        pltpu.SemaphoreType.DMA,
    ],
)
def _s_call(rows_hbm, cols_hbm, st_hbm, t_hbm,
            rows_v, cols_v, st_v, bkt_v, val_v, sem):
    _, base_w = _worker_base()

    def chunk(g, carry):
        off = base_w + g * CHUNK
        pltpu.sync_copy(rows_hbm.at[pl.ds(off, CHUNK)], rows_v)
        pltpu.sync_copy(cols_hbm.at[pl.ds(off, CHUNK)], cols_v)
        pltpu.sync_copy(st_hbm.at[pl.ds(off, CHUNK)], st_v)
        for j in range(16):
            for i in range(8):
                o = j * 128 + i * L
                low7, bkt = _keys(rows_v, cols_v, o)
                act = st_v[pl.ds(o, L)] == 1
                bkt = lax.select(act, bkt, jnp.full((L,), DUMMY, jnp.int32))
                eid = lax.iota(jnp.int32, L) + (off + o)
                val = lax.bitwise_or(lax.shift_left(eid, jnp.int32(7)), low7)
                bkt_v[jnp.int32(j), pl.ds(i * L, L)] = bkt
                val_v[jnp.int32(j), pl.ds(i * L, L)] = val
        copies = [
            pltpu.async_copy(val_v.at[jnp.int32(j)], t_hbm.at[bkt_v.at[jnp.int32(j)]], sem)
            for j in range(16)
        ]
        for cp in copies:
            cp.wait()
        return carry

    lax.fori_loop(jnp.int32(0), jnp.int32(CHUNKS_PER_W), chunk, jnp.int32(0))


@functools.partial(
    pl.kernel,
    out_type=[
        jax.ShapeDtypeStruct((NE2,), jnp.int32),
        jax.ShapeDtypeStruct((NW, L), jnp.int32),
    ],
    mesh=_MESH,
    scratch_types=[
        pltpu.VMEM((CHUNK,), jnp.int32),
        pltpu.VMEM((CHUNK,), jnp.int32),
        pltpu.VMEM((CHUNK,), jnp.int32),
        pltpu.VMEM((16, 128), jnp.int32),
        pltpu.VMEM((16, 128), jnp.int32),
        pltpu.VMEM((CHUNK,), jnp.int32),
        pltpu.VMEM((L,), jnp.int32),
        pltpu.SemaphoreType.DMA,
    ],
)
def _r_call(rows_hbm, cols_hbm, st_hbm, t_hbm, stn_hbm, cnt_hbm,
            rows_v, cols_v, st_v, bkt_v, tv_v, stn_v, cnt_v, sem):
    w, base_w = _worker_base()

    def chunk(g, cnt):
        off = base_w + g * CHUNK
        acc = jnp.zeros((L,), jnp.int32)
        pltpu.sync_copy(rows_hbm.at[pl.ds(off, CHUNK)], rows_v)
        pltpu.sync_copy(cols_hbm.at[pl.ds(off, CHUNK)], cols_v)
        pltpu.sync_copy(st_hbm.at[pl.ds(off, CHUNK)], st_v)
        for j in range(16):
            for i in range(8):
                o = j * 128 + i * L
                _, bkt = _keys(rows_v, cols_v, o)
                act = st_v[pl.ds(o, L)] == 1
                bkt = lax.select(act, bkt, jnp.full((L,), DUMMY, jnp.int32))
                bkt_v[jnp.int32(j), pl.ds(i * L, L)] = bkt
        copies = [
            pltpu.async_copy(t_hbm.at[bkt_v.at[jnp.int32(j)]], tv_v.at[jnp.int32(j)], sem)
            for j in range(16)
        ]
        for cp in copies:
            cp.wait()
        for j in range(16):
            for i in range(8):
                o = j * 128 + i * L
                low7, _ = _keys(rows_v, cols_v, o)
                st = st_v[pl.ds(o, L)]
                tv = tv_v[jnp.int32(j), pl.ds(i * L, L)]
                act = st == 1
                match = jnp.logical_and(
                    act, lax.bitwise_and(tv, jnp.int32(127)) == low7)
                eid = lax.iota(jnp.int32, L) + (off + o)
                win = jnp.logical_and(
                    match, lax.shift_right_logical(tv, jnp.int32(7)) == eid)
                newst = jnp.where(
                    match,
                    jnp.where(win, jnp.full((L,), 2, jnp.int32),
                              jnp.full((L,), 0, jnp.int32)),
                    st)
                stn_v[pl.ds(o, L)] = newst
                acc = acc + lax.select(newst == jnp.int32(1),
                                       jnp.full((L,), 1, jnp.int32),
                                       jnp.full((L,), 0, jnp.int32))
        pltpu.sync_copy(stn_v, stn_hbm.at[pl.ds(off, CHUNK)])
        cnt_v[...] = cnt_v[...] + acc
        return cnt

    cnt_v[...] = jnp.zeros((L,), jnp.int32)
    lax.fori_loop(jnp.int32(0), jnp.int32(CHUNKS_PER_W), chunk,
                  jnp.int32(0))
    pltpu.sync_copy(cnt_v, cnt_hbm.at[w])


@functools.partial(
    pl.kernel,
    out_type=jax.ShapeDtypeStruct((NC, NPAD_NODES), jnp.float32),
    mesh=_MESH,
    scratch_types=[
        pltpu.VMEM((CHUNK,), jnp.int32),
        pltpu.VMEM((CHUNK,), jnp.int32),
        pltpu.VMEM((16, 128), jnp.int32),
        pltpu.VMEM((16, 128), jnp.float32),
        pltpu.VMEM((DEG_SLICE,), jnp.float32),
        pltpu.VMEM_SHARED((NPAD_NODES,), jnp.float32),
        pltpu.SemaphoreType.DMA,
    ],
)
def _d_call(rows_hbm, st_hbm, degp_hbm,
            rows_v, st_v, ridx_v, add_v, z_v, deg_sh, sem):
    c = lax.axis_index("c")
    s = lax.axis_index("s")
    base_w = (c * NS + s) * PER_W

    def zb(i, carry):
        z_v[pl.ds(i * L, L)] = jnp.zeros((L,), jnp.float32)
        return carry

    lax.fori_loop(jnp.int32(0), jnp.int32(DEG_SLICE // L), zb, jnp.int32(0))
    pltpu.sync_copy(z_v, deg_sh.at[pl.ds(s * DEG_SLICE, DEG_SLICE)])
    plsc.subcore_barrier()

    def chunk(g, carry):
        off = base_w + g * CHUNK
        pltpu.sync_copy(rows_hbm.at[pl.ds(off, CHUNK)], rows_v)
        pltpu.sync_copy(st_hbm.at[pl.ds(off, CHUNK)], st_v)
        for j in range(16):
            for i in range(8):
                o = j * 128 + i * L
                r = rows_v[pl.ds(o, L)]
                st = st_v[pl.ds(o, L)]
                ridx_v[jnp.int32(j), pl.ds(i * L, L)] = r
                add_v[jnp.int32(j), pl.ds(i * L, L)] = jnp.where(
                    st == 2, jnp.float32(1.0), jnp.float32(0.0))
        copies = [
            pltpu.async_copy(add_v.at[jnp.int32(j)], deg_sh.at[ridx_v.at[jnp.int32(j)]], sem,
                             add=True)
            for j in range(16)
        ]
        for cp in copies:
            cp.wait()
        return carry

    lax.fori_loop(jnp.int32(0), jnp.int32(CHUNKS_PER_W), chunk, jnp.int32(0))
    plsc.subcore_barrier()
    pltpu.sync_copy(deg_sh.at[pl.ds(s * DEG_SLICE, DEG_SLICE)],
                    degp_hbm.at[c, pl.ds(s * DEG_SLICE, DEG_SLICE)])


def _pe_body(degp_ref, out_ref):
    dp = degp_ref[...]
    deg = dp[0, :] + dp[1, :]
    deginv = 1.0 / (deg + jnp.float32(1e-8))
    k = (lax.broadcasted_iota(jnp.int32, (1, PE_DIM), 1) + 1
         ).astype(jnp.float32)
    t = deginv[:, None] * k
    t = t * jnp.float32(math.pi)
    out_ref[...] = jnp.clip(jnp.sin(t), -2.0, 2.0)


def _pe_call(degp):
    return pl.pallas_call(
        _pe_body,
        grid=(NPAD_NODES // PE_BLK,),
        in_specs=[pl.BlockSpec((NC, PE_BLK), lambda i: (jnp.int32(0), i))],
        out_specs=pl.BlockSpec((PE_BLK, PE_DIM),
                               lambda i: (i, jnp.int32(0))),
        out_shape=jax.ShapeDtypeStruct((NPAD_NODES, PE_DIM), jnp.float32),
    )(degp)


def kernel(edge_index, num_nodes):
    del num_nodes
    ei = edge_index.astype(jnp.int32)
    rows = jnp.pad(jnp.concatenate([ei[0], ei[1]]), (0, PAD))
    cols = jnp.pad(jnp.concatenate([ei[1], ei[0]]), (0, PAD))
    state0 = jnp.pad(jnp.ones((NE2_RAW,), jnp.int32), (0, PAD))

    def cond(carry):
        _, cnt, r = carry
        return jnp.logical_and(cnt > 0, r < 130)

    def body(carry):
        state, _, r = carry
        tbl = _s_call(rows, cols, state)
        state2, cnts = _r_call(rows, cols, state, tbl)
        return state2, jnp.sum(cnts, dtype=jnp.int32), r + 1

    state_f, _, _ = lax.while_loop(
        cond, body, (state0, jnp.int32(NE2_RAW), jnp.int32(0)))
    degp = _d_call(rows, state_f)
    pe = _pe_call(degp)
    return pe[:N_NODES]


# speedup vs baseline: 48.7253x; 48.7253x over previous
import functools
import math

import jax
import jax.numpy as jnp
from jax import lax
from jax.experimental import pallas as pl
from jax.experimental.pallas import tpu as pltpu
from jax.experimental.pallas import tpu_sc as plsc

N_NODES = 100000
PE_DIM = 64
NE2_RAW = 6400000
NC, NS, L = 2, 16, 16
NW = NC * NS
CHUNK = 2048
CHUNKS_PER_W = 98
PER_W = CHUNK * CHUNKS_PER_W
NE2 = NW * PER_W
PAD = NE2 - NE2_RAW

NB_USED = 78125000
DUMMY = NB_USED
DUMMY_SPREAD = 32768
NB = NB_USED + DUMMY_SPREAD

NPAD_NODES = 102400
DEG_SLICE = NPAD_NODES // NS
PE_BLK = 2048

_MESH = plsc.VectorSubcoreMesh(core_axis_name="c", subcore_axis_name="s")


def _keys(rows_v, cols_v, o):
    r = rows_v[pl.ds(o, L)]
    c = cols_v[pl.ds(o, L)]
    t = r * 32 + c
    low7 = lax.bitwise_and(t, jnp.int32(127))
    bkt = r * 781 + lax.shift_right_logical(t, jnp.int32(7))
    return low7, bkt


def _worker_base():
    w = lax.axis_index("c") * NS + lax.axis_index("s")
    return w, w * PER_W


@functools.partial(
    pl.kernel,
    out_type=jax.ShapeDtypeStruct((NB,), jnp.int32),
    mesh=_MESH,
    scratch_types=[
        pltpu.VMEM((CHUNK,), jnp.int32),
        pltpu.VMEM((CHUNK,), jnp.int32),
        pltpu.VMEM((CHUNK,), jnp.int32),
        pltpu.VMEM((CHUNK,), jnp.int32),
        pltpu.VMEM((CHUNK,), jnp.int32),
        pltpu.SemaphoreType.DMA,
    ],
)
def _s_call(rows_hbm, cols_hbm, st_hbm, t_hbm,
            rows_v, cols_v, st_v, bkt_v, val_v, sem):
    _, base_w = _worker_base()

    def chunk(g, carry):
        off = base_w + g * CHUNK
        pltpu.sync_copy(rows_hbm.at[pl.ds(off, CHUNK)], rows_v)
        pltpu.sync_copy(cols_hbm.at[pl.ds(off, CHUNK)], cols_v)
        pltpu.sync_copy(st_hbm.at[pl.ds(off, CHUNK)], st_v)
        for j in range(16):
            for i in range(8):
                o = j * 128 + i * L
                low7, bkt = _keys(rows_v, cols_v, o)
                act = st_v[pl.ds(o, L)] == 1
                eid = lax.iota(jnp.int32, L) + (off + o)
                park = jnp.int32(DUMMY) + lax.bitwise_and(
                    eid, jnp.int32(DUMMY_SPREAD - 1))
                bkt = lax.select(act, bkt, park)
                val = lax.bitwise_or(lax.shift_left(eid, jnp.int32(7)), low7)
                bkt_v[pl.ds(o, L)] = bkt
                val_v[pl.ds(o, L)] = val
        pltpu.async_copy(val_v, t_hbm.at[bkt_v], sem).wait()
        return carry

    lax.fori_loop(jnp.int32(0), jnp.int32(CHUNKS_PER_W), chunk, jnp.int32(0))


@functools.partial(
    pl.kernel,
    out_type=[
        jax.ShapeDtypeStruct((NE2,), jnp.int32),
        jax.ShapeDtypeStruct((NW, L), jnp.int32),
    ],
    mesh=_MESH,
    scratch_types=[
        pltpu.VMEM((CHUNK,), jnp.int32),
        pltpu.VMEM((CHUNK,), jnp.int32),
        pltpu.VMEM((CHUNK,), jnp.int32),
        pltpu.VMEM((CHUNK,), jnp.int32),
        pltpu.VMEM((CHUNK,), jnp.int32),
        pltpu.VMEM((CHUNK,), jnp.int32),
        pltpu.VMEM((L,), jnp.int32),
        pltpu.SemaphoreType.DMA,
    ],
)
def _r_call(rows_hbm, cols_hbm, st_hbm, t_hbm, stn_hbm, cnt_hbm,
            rows_v, cols_v, st_v, bkt_v, tv_v, stn_v, cnt_v, sem):
    w, base_w = _worker_base()

    def chunk(g, cnt):
        off = base_w + g * CHUNK
        acc = jnp.zeros((L,), jnp.int32)
        pltpu.sync_copy(rows_hbm.at[pl.ds(off, CHUNK)], rows_v)
        pltpu.sync_copy(cols_hbm.at[pl.ds(off, CHUNK)], cols_v)
        pltpu.sync_copy(st_hbm.at[pl.ds(off, CHUNK)], st_v)
        for j in range(16):
            for i in range(8):
                o = j * 128 + i * L
                _, bkt = _keys(rows_v, cols_v, o)
                act = st_v[pl.ds(o, L)] == 1
                eid = lax.iota(jnp.int32, L) + (off + o)
                park = jnp.int32(DUMMY) + lax.bitwise_and(
                    eid, jnp.int32(DUMMY_SPREAD - 1))
                bkt = lax.select(act, bkt, park)
                bkt_v[pl.ds(o, L)] = bkt
        pltpu.async_copy(t_hbm.at[bkt_v], tv_v, sem).wait()
        for j in range(16):
            for i in range(8):
                o = j * 128 + i * L
                low7, _ = _keys(rows_v, cols_v, o)
                st = st_v[pl.ds(o, L)]
                tv = tv_v[pl.ds(o, L)]
                act = st == 1
                match = jnp.logical_and(
                    act, lax.bitwise_and(tv, jnp.int32(127)) == low7)
                eid = lax.iota(jnp.int32, L) + (off + o)
                win = jnp.logical_and(
                    match, lax.shift_right_logical(tv, jnp.int32(7)) == eid)
                newst = jnp.where(
                    match,
                    jnp.where(win, jnp.full((L,), 2, jnp.int32),
                              jnp.full((L,), 0, jnp.int32)),
                    st)
                stn_v[pl.ds(o, L)] = newst
                acc = acc + lax.select(newst == jnp.int32(1),
                                       jnp.full((L,), 1, jnp.int32),
                                       jnp.full((L,), 0, jnp.int32))
        pltpu.sync_copy(stn_v, stn_hbm.at[pl.ds(off, CHUNK)])
        cnt_v[...] = cnt_v[...] + acc
        return cnt

    cnt_v[...] = jnp.zeros((L,), jnp.int32)
    lax.fori_loop(jnp.int32(0), jnp.int32(CHUNKS_PER_W), chunk,
                  jnp.int32(0))
    pltpu.sync_copy(cnt_v, cnt_hbm.at[w])


@functools.partial(
    pl.kernel,
    out_type=jax.ShapeDtypeStruct((NC, NPAD_NODES), jnp.float32),
    mesh=_MESH,
    scratch_types=[
        pltpu.VMEM((CHUNK,), jnp.int32),
        pltpu.VMEM((CHUNK,), jnp.int32),
        pltpu.VMEM((CHUNK,), jnp.int32),
        pltpu.VMEM((CHUNK,), jnp.float32),
        pltpu.VMEM((DEG_SLICE,), jnp.float32),
        pltpu.VMEM_SHARED((NPAD_NODES,), jnp.float32),
        pltpu.SemaphoreType.DMA,
    ],
)
def _d_call(rows_hbm, st_hbm, degp_hbm,
            rows_v, st_v, ridx_v, add_v, z_v, deg_sh, sem):
    c = lax.axis_index("c")
    s = lax.axis_index("s")
    base_w = (c * NS + s) * PER_W

    def zb(i, carry):
        z_v[pl.ds(i * L, L)] = jnp.zeros((L,), jnp.float32)
        return carry

    lax.fori_loop(jnp.int32(0), jnp.int32(DEG_SLICE // L), zb, jnp.int32(0))
    pltpu.sync_copy(z_v, deg_sh.at[pl.ds(s * DEG_SLICE, DEG_SLICE)])
    plsc.subcore_barrier()

    def chunk(g, carry):
        off = base_w + g * CHUNK
        pltpu.sync_copy(rows_hbm.at[pl.ds(off, CHUNK)], rows_v)
        pltpu.sync_copy(st_hbm.at[pl.ds(off, CHUNK)], st_v)
        for j in range(16):
            for i in range(8):
                o = j * 128 + i * L
                r = rows_v[pl.ds(o, L)]
                st = st_v[pl.ds(o, L)]
                ridx_v[pl.ds(o, L)] = r
                add_v[pl.ds(o, L)] = jnp.where(
                    st == 2, jnp.float32(1.0), jnp.float32(0.0))
        pltpu.async_copy(add_v, deg_sh.at[ridx_v], sem, add=True).wait()
        return carry

    lax.fori_loop(jnp.int32(0), jnp.int32(CHUNKS_PER_W), chunk, jnp.int32(0))
    plsc.subcore_barrier()
    pltpu.sync_copy(deg_sh.at[pl.ds(s * DEG_SLICE, DEG_SLICE)],
                    degp_hbm.at[c, pl.ds(s * DEG_SLICE, DEG_SLICE)])


def _pe_body(degp_ref, out_ref):
    dp = degp_ref[...]
    deg = dp[0, :] + dp[1, :]
    deginv = 1.0 / (deg + jnp.float32(1e-8))
    k = (lax.broadcasted_iota(jnp.int32, (1, PE_DIM), 1) + 1
         ).astype(jnp.float32)
    t = deginv[:, None] * k
    t = t * jnp.float32(math.pi)
    out_ref[...] = jnp.clip(jnp.sin(t), -2.0, 2.0)


def _pe_call(degp):
    return pl.pallas_call(
        _pe_body,
        grid=(NPAD_NODES // PE_BLK,),
        in_specs=[pl.BlockSpec((NC, PE_BLK), lambda i: (jnp.int32(0), i))],
        out_specs=pl.BlockSpec((PE_BLK, PE_DIM),
                               lambda i: (i, jnp.int32(0))),
        out_shape=jax.ShapeDtypeStruct((NPAD_NODES, PE_DIM), jnp.float32),
    )(degp)


def kernel(edge_index, num_nodes):
    del num_nodes
    ei = edge_index.astype(jnp.int32)
    rows = jnp.pad(jnp.concatenate([ei[0], ei[1]]), (0, PAD))
    cols = jnp.pad(jnp.concatenate([ei[1], ei[0]]), (0, PAD))
    state0 = jnp.pad(jnp.ones((NE2_RAW,), jnp.int32), (0, PAD))

    def cond(carry):
        _, cnt, r = carry
        return jnp.logical_and(cnt > 0, r < 130)

    def body(carry):
        state, _, r = carry
        tbl = _s_call(rows, cols, state)
        state2, cnts = _r_call(rows, cols, state, tbl)
        return state2, jnp.sum(cnts, dtype=jnp.int32), r + 1

    state_f, _, _ = lax.while_loop(
        cond, body, (state0, jnp.int32(NE2_RAW), jnp.int32(0)))
    degp = _d_call(rows, state_f)
    pe = _pe_call(degp)
    return pe[:N_NODES]


# speedup vs baseline: 438.3885x; 8.9971x over previous
import functools
import math

import jax
import jax.numpy as jnp
from jax import lax
from jax.experimental import pallas as pl
from jax.experimental.pallas import tpu as pltpu
from jax.experimental.pallas import tpu_sc as plsc

N_NODES = 100000
PE_DIM = 64
NE2_RAW = 6400000
NC, NS, L = 2, 16, 16
CHUNK = 8192
SUB = 256
TRIPS1 = 50
PER_T = CHUNK * TRIPS1
NE2 = NS * PER_T
PAD = NE2 - NE2_RAW
SENT = 100000

HALF_B = 610352
PARK_N = 8192
TBL_N = HALF_B + PARK_N
ID_OFF = 8192

SCAP = 212992
SBLK = SCAP // CHUNK
WBLK = 40
WCAP = WBLK * CHUNK
NPAD_NODES = 102400
DEG_SLICE = NPAD_NODES // NS
PE_BLK = 2048

_MESH = plsc.VectorSubcoreMesh(core_axis_name="c", subcore_axis_name="s")


def _keys16(r, c):
    t = r * 1696 + c
    low = lax.bitwise_and(t, jnp.int32(8191))
    bkt = r * 12 + lax.shift_right_logical(t, jnp.int32(13))
    return low, bkt


@functools.partial(
    pl.kernel,
    out_type=[
        jax.ShapeDtypeStruct((NC * NS * WCAP,), jnp.int32),
        jax.ShapeDtypeStruct((NC * NS * 2 * SCAP,), jnp.int32),
        jax.ShapeDtypeStruct((NC * NS * 2 * SCAP,), jnp.int32),
    ],
    mesh=_MESH,
    compiler_params=pltpu.CompilerParams(needs_layout_passes=False),
    scratch_types=[
        pltpu.VMEM((CHUNK,), jnp.int32),
        pltpu.VMEM((CHUNK,), jnp.int32),
        pltpu.VMEM((CHUNK,), jnp.int32),
        pltpu.VMEM((CHUNK,), jnp.int32),
        pltpu.VMEM((CHUNK,), jnp.int32),
        pltpu.VMEM((2 * CHUNK,), jnp.int32),
        pltpu.VMEM((2 * CHUNK,), jnp.int32),
        pltpu.VMEM((2 * CHUNK,), jnp.int32),
        pltpu.VMEM((L,), jnp.int32),
        pltpu.VMEM((NS * L,), jnp.int32),
        pltpu.VMEM_SHARED((TBL_N,), jnp.int32),
        pltpu.VMEM_SHARED((NS * L,), jnp.int32),
        pltpu.SemaphoreType.DMA,
    ],
)
def _elect(rows_hbm, cols_hbm, wr_hbm, sr_hbm, sc_hbm,
           rows_v, cols_v, bkt_v, val_v, tv_v,
           stage_r, stage_c, wstage, cnt_v, comm_v,
           tbl_sh, comm_sh, sem):
    c = lax.axis_index("c")
    s = lax.axis_index("s")
    half_lo = c * jnp.int32(HALF_B)

    scr0 = ((c * NS + s) * 2) * jnp.int32(SCAP)
    scr1 = scr0 + jnp.int32(SCAP)
    wbase = (c * NS + s) * jnp.int32(WCAP)

    def wflush_last(wcur):
        comp = lax.shift_right_logical(wcur, jnp.int32(13))
        last = jnp.clip(comp - jnp.int32(1), jnp.int32(0),
                        jnp.int32(WBLK - 1))
        half = lax.bitwise_and(last, jnp.int32(1))
        hoff = half * jnp.int32(CHUNK)
        doff = wbase + last * jnp.int32(CHUNK)
        pltpu.sync_copy(wstage.at[pl.ds(hoff, CHUNK)],
                        wr_hbm.at[pl.ds(doff, CHUNK)])

    def load_chunk(first, pp, g):
        if first:
            o1 = s * PER_T + g * CHUNK
            pltpu.sync_copy(rows_hbm.at[pl.ds(o1, CHUNK)], rows_v)
            pltpu.sync_copy(cols_hbm.at[pl.ds(o1, CHUNK)], cols_v)
        else:
            src = jnp.where(pp == 0, scr0, scr1) + g * CHUNK
            pltpu.sync_copy(sr_hbm.at[pl.ds(src, CHUNK)], rows_v)
            pltpu.sync_copy(sc_hbm.at[pl.ds(src, CHUNK)], cols_v)

    def idbase(first, g):
        return (s * PER_T if first else s * SCAP) + g * CHUNK

    def lane_info(o, base):
        r = rows_v[pl.ds(o, L)]
        cc = cols_v[pl.ds(o, L)]
        low, bkt = _keys16(r, cc)
        lb = bkt - half_lo
        ids = lax.iota(jnp.int32, L) + (base + o) + jnp.int32(ID_OFF)
        parked = jnp.logical_or(
            r >= jnp.int32(SENT),
            jnp.logical_or(lb < jnp.int32(0), lb >= jnp.int32(HALF_B)))
        park = jnp.int32(HALF_B) + lax.bitwise_and(
            ids, jnp.int32(PARK_N - 1))
        addr = lax.select(parked, park, lb)
        return r, cc, low, ids, parked, addr

    def make_p1(first):
      def p1_chunk(g, carry):
        pp = carry
        load_chunk(first, pp, g)
        base = idbase(first, g)

        def sub(k, cr):
            for q in range(SUB // L):
                o = k * SUB + q * L
                _, _, _, ids, _, addr = lane_info(o, base)
                bkt_v[pl.ds(o, L)] = addr
                val_v[pl.ds(o, L)] = ids
            return cr

        lax.fori_loop(jnp.int32(0), jnp.int32(CHUNK // SUB), sub,
                      jnp.int32(0))
        pltpu.sync_copy(val_v, tbl_sh.at[bkt_v])
        return carry
      return p1_chunk

    def make_p2(first):
      def p2_chunk(g, carry):
        pp, wcur = carry
        load_chunk(first, pp, g)
        base = idbase(first, g)

        def subi(k, cr):
            for q in range(SUB // L):
                o = k * SUB + q * L
                _, _, _, _, _, addr = lane_info(o, base)
                bkt_v[pl.ds(o, L)] = addr
            return cr

        lax.fori_loop(jnp.int32(0), jnp.int32(CHUNK // SUB), subi,
                      jnp.int32(0))
        pltpu.sync_copy(tbl_sh.at[bkt_v], tv_v)

        def subw(k, wcur):
            for q in range(SUB // L):
                o = k * SUB + q * L
                r, _, low, ids, parked, addr = lane_info(o, base)
                tv = tv_v[pl.ds(o, L)]
                win = jnp.logical_and(jnp.logical_not(parked), tv == ids)
                wint = lax.select(win, jnp.full((L,), 1, jnp.int32),
                                  jnp.full((L,), 0, jnp.int32))
                rank = plsc.cumsum(wint) - wint
                idx = lax.bitwise_and(wcur + rank,
                                      jnp.int32(2 * CHUNK - 1))
                plsc.store_scatter(wstage, [idx], r, mask=win)
                wcur = wcur + jnp.sum(wint, dtype=jnp.int32)
                park = jnp.int32(HALF_B) + lax.bitwise_and(
                    ids, jnp.int32(PARK_N - 1))
                bkt_v[pl.ds(o, L)] = lax.select(win, addr, park)
                val_v[pl.ds(o, L)] = low
            return wcur

        wcur = lax.fori_loop(jnp.int32(0), jnp.int32(CHUNK // SUB), subw,
                             wcur)
        pltpu.sync_copy(val_v, tbl_sh.at[bkt_v])
        wflush_last(wcur)
        return pp, wcur
      return p2_chunk

    def flush_last(cur, pp):
        dstb = jnp.where(pp == 0, scr1, scr0)
        comp = lax.shift_right_logical(cur, jnp.int32(13))
        last = jnp.clip(comp - jnp.int32(1), jnp.int32(0),
                        jnp.int32(SBLK - 1))
        half = lax.bitwise_and(last, jnp.int32(1))
        hoff = half * jnp.int32(CHUNK)
        doff = dstb + last * jnp.int32(CHUNK)
        pltpu.sync_copy(stage_r.at[pl.ds(hoff, CHUNK)],
                        sr_hbm.at[pl.ds(doff, CHUNK)])
        pltpu.sync_copy(stage_c.at[pl.ds(hoff, CHUNK)],
                        sc_hbm.at[pl.ds(doff, CHUNK)])

    def make_p3(first):
      def p3_chunk(g, carry):
        pp, cur = carry
        load_chunk(first, pp, g)
        base = idbase(first, g)

        def subi(k, cr):
            for q in range(SUB // L):
                o = k * SUB + q * L
                _, _, _, _, _, addr = lane_info(o, base)
                bkt_v[pl.ds(o, L)] = addr
            return cr

        lax.fori_loop(jnp.int32(0), jnp.int32(CHUNK // SUB), subi,
                      jnp.int32(0))
        pltpu.sync_copy(tbl_sh.at[bkt_v], tv_v)

        def subs(k, cur):
            for q in range(SUB // L):
                o = k * SUB + q * L
                r, cc, low, _, parked, _ = lane_info(o, base)
                tv = tv_v[pl.ds(o, L)]
                gone = jnp.logical_or(parked, tv == low)
                sint = lax.select(gone, jnp.full((L,), 0, jnp.int32),
                                  jnp.full((L,), 1, jnp.int32))
                still = sint == jnp.int32(1)
                rank = plsc.cumsum(sint) - sint
                idx = lax.bitwise_and(cur + rank,
                                      jnp.int32(2 * CHUNK - 1))
                plsc.store_scatter(stage_r, [idx], r, mask=still)
                plsc.store_scatter(stage_c, [idx], cc, mask=still)
                cur = cur + jnp.sum(sint, dtype=jnp.int32)
            return cur

        cur = lax.fori_loop(jnp.int32(0), jnp.int32(CHUNK // SUB),
                            subs, cur)
        flush_last(cur, pp)
        return pp, cur
      return p3_chunk

    def finish_round(cur, pp):
        real = cur

        def pad_cond(cr):
            return lax.bitwise_and(cr, jnp.int32(CHUNK - 1)) != jnp.int32(0)

        def pad_body(cr):
            rem = jnp.int32(CHUNK) - lax.bitwise_and(
                cr, jnp.int32(CHUNK - 1))
            n = jnp.minimum(rem, jnp.int32(L))
            m = lax.iota(jnp.int32, L) < n
            idx = lax.bitwise_and(cr + lax.iota(jnp.int32, L),
                                  jnp.int32(2 * CHUNK - 1))
            plsc.store_scatter(stage_r, [idx],
                               jnp.full((L,), SENT, jnp.int32), mask=m)
            plsc.store_scatter(stage_c, [idx],
                               jnp.zeros((L,), jnp.int32), mask=m)
            return cr + n

        cur = lax.while_loop(pad_cond, pad_body, cur)
        flush_last(cur, pp)

        cnt_v[...] = jnp.full((L,), real, jnp.int32)
        pltpu.sync_copy(cnt_v, comm_sh.at[pl.ds(s * L, L)])
        plsc.subcore_barrier()
        pltpu.sync_copy(comm_sh, comm_v)
        tot = jnp.zeros((L,), jnp.int32)
        for t in range(NS):
            tot = tot + comm_v[pl.ds(t * L, L)]
        total = jnp.sum(tot, dtype=jnp.int32) // jnp.int32(L)
        return total, cur

    lax.fori_loop(jnp.int32(0), jnp.int32(TRIPS1), make_p1(True),
                  jnp.int32(0))
    plsc.subcore_barrier()
    _, wcur = lax.fori_loop(jnp.int32(0), jnp.int32(TRIPS1), make_p2(True),
                            (jnp.int32(0), jnp.int32(0)))
    plsc.subcore_barrier()
    _, cur = lax.fori_loop(jnp.int32(0), jnp.int32(TRIPS1), make_p3(True),
                           (jnp.int32(0), jnp.int32(0)))
    total, cur = finish_round(cur, jnp.int32(0))

    def round_body(carry):
        _, srccnt, pp, wcur = carry
        trips = lax.shift_right_logical(srccnt, jnp.int32(13))
        lax.fori_loop(jnp.int32(0), trips, make_p1(False), pp)
        plsc.subcore_barrier()
        _, wcur = lax.fori_loop(jnp.int32(0), trips, make_p2(False),
                                (pp, wcur))
        plsc.subcore_barrier()
        _, cur = lax.fori_loop(jnp.int32(0), trips, make_p3(False),
                               (pp, jnp.int32(0)))
        total, cur = finish_round(cur, pp)
        return total, cur, jnp.int32(1) - pp, wcur

    def round_cond2(carry):
        total, _, _, _ = carry
        return total > jnp.int32(0)

    _, _, _, wcur = lax.while_loop(round_cond2, round_body,
                                   (total, cur, jnp.int32(1), wcur))

    def wpad_cond(cr):
        return lax.bitwise_and(cr, jnp.int32(CHUNK - 1)) != jnp.int32(0)

    def wpad_body(cr):
        rem = jnp.int32(CHUNK) - lax.bitwise_and(cr, jnp.int32(CHUNK - 1))
        n = jnp.minimum(rem, jnp.int32(L))
        m = lax.iota(jnp.int32, L) < n
        idx = lax.bitwise_and(cr + lax.iota(jnp.int32, L),
                              jnp.int32(2 * CHUNK - 1))
        plsc.store_scatter(wstage, [idx],
                           jnp.full((L,), SENT, jnp.int32), mask=m)
        return cr + n

    wcur = lax.while_loop(wpad_cond, wpad_body, wcur)
    wflush_last(wcur)

    def sfill(i, carry):
        wstage[pl.ds(i * L, L)] = jnp.full((L,), SENT, jnp.int32)
        return carry

    lax.fori_loop(jnp.int32(0), jnp.int32(CHUNK // L), sfill, jnp.int32(0))

    def sflush(b, carry):
        pltpu.sync_copy(wstage.at[pl.ds(0, CHUNK)],
                        wr_hbm.at[pl.ds(wbase + b * jnp.int32(CHUNK),
                                        CHUNK)])
        return carry

    comp = lax.shift_right_logical(wcur, jnp.int32(13))
    lax.fori_loop(comp, jnp.int32(WBLK), sflush, jnp.int32(0))


@functools.partial(
    pl.kernel,
    out_type=jax.ShapeDtypeStruct((NC, NPAD_NODES), jnp.float32),
    mesh=_MESH,
    compiler_params=pltpu.CompilerParams(needs_layout_passes=False),
    scratch_types=[
        pltpu.VMEM((CHUNK,), jnp.int32),
        pltpu.VMEM((CHUNK,), jnp.float32),
        pltpu.VMEM((DEG_SLICE,), jnp.float32),
        pltpu.VMEM_SHARED((NPAD_NODES,), jnp.float32),
        pltpu.SemaphoreType.DMA,
    ],
)
def _dsum(wr_hbm, degp_hbm, wrow_v, dadd_v, z_v, deg_sh, sem):
    c = lax.axis_index("c")
    s = lax.axis_index("s")

    def zb(i, carry):
        z_v[pl.ds(i * L, L)] = jnp.zeros((L,), jnp.float32)
        return carry

    lax.fori_loop(jnp.int32(0), jnp.int32(DEG_SLICE // L), zb, jnp.int32(0))
    pltpu.sync_copy(z_v, deg_sh.at[pl.ds(s * DEG_SLICE, DEG_SLICE)])
    plsc.subcore_barrier()

    wbase = (c * NS + s) * jnp.int32(WCAP)

    def chunk(g, carry):
        pltpu.sync_copy(wr_hbm.at[pl.ds(wbase + g * CHUNK, CHUNK)], wrow_v)

        def sub(k, cr):
            for q in range(SUB // L):
                o = k * SUB + q * L
                r = wrow_v[pl.ds(o, L)]
                dadd_v[pl.ds(o, L)] = lax.select(
                    r < jnp.int32(SENT),
                    jnp.full((L,), 1.0, jnp.float32),
                    jnp.full((L,), 0.0, jnp.float32))
            return cr

        lax.fori_loop(jnp.int32(0), jnp.int32(CHUNK // SUB), sub,
                      jnp.int32(0))
        pltpu.sync_copy(dadd_v, deg_sh.at[wrow_v], add=True)
        return carry

    lax.fori_loop(jnp.int32(0), jnp.int32(WBLK), chunk, jnp.int32(0))
    plsc.subcore_barrier()
    pltpu.sync_copy(deg_sh.at[pl.ds(s * DEG_SLICE, DEG_SLICE)],
                    degp_hbm.at[c, pl.ds(s * DEG_SLICE, DEG_SLICE)])


def _pe_body(degp_ref, out_ref):
    dp = degp_ref[...]
    deg = dp[0, :] + dp[1, :]
    deginv = 1.0 / (deg + jnp.float32(1e-8))
    k = (lax.broadcasted_iota(jnp.int32, (1, PE_DIM), 1) + 1
         ).astype(jnp.float32)
    t = deginv[:, None] * k
    t = t * jnp.float32(math.pi)
    out_ref[...] = jnp.clip(jnp.sin(t), -2.0, 2.0)


def _pe_call(degp):
    return pl.pallas_call(
        _pe_body,
        grid=(NPAD_NODES // PE_BLK,),
        in_specs=[pl.BlockSpec((NC, PE_BLK), lambda i: (jnp.int32(0), i))],
        out_specs=pl.BlockSpec((PE_BLK, PE_DIM),
                               lambda i: (i, jnp.int32(0))),
        out_shape=jax.ShapeDtypeStruct((NPAD_NODES, PE_DIM), jnp.float32),
    )(degp)


def kernel(edge_index, num_nodes):
    del num_nodes
    ei = edge_index.astype(jnp.int32)
    rows = jnp.pad(jnp.concatenate([ei[0], ei[1]]), (0, PAD),
                   constant_values=SENT)
    cols = jnp.pad(jnp.concatenate([ei[1], ei[0]]), (0, PAD))
    wrows, _, _ = _elect(rows, cols)
    degp = _dsum(wrows)
    pe = _pe_call(degp)
    return pe[:N_NODES]
